# baseline (speedup 1.0000x reference)
import functools
import math

import jax
import jax.numpy as jnp
import numpy as np
from jax import lax
from jax.experimental import pallas as pl
from jax.experimental.pallas import tpu as pltpu
from jax.experimental.pallas import tpu_sc as plsc

D_MODEL = 1024
POS_DIM = 16
B, L = 4, 4096
N_TOK = B * L
NC, NS, LANES = 2, 16, 16
NW = NC * NS
TOK_PER_W = N_TOK // NW
L_PER_W = L // NW

TC_BLK = 4096
N_LBLK = L // TC_BLK


def _pe_table():
    position = np.arange(L)[:, None].astype(np.float32)
    div_term = np.exp(
        np.arange(0, D_MODEL, 2).astype(np.float32) * (-math.log(10000.0) / D_MODEL)
    )
    pe = np.zeros((L, D_MODEL), dtype=np.float32)
    pe[:, 0::2] = np.sin(position * div_term)
    pe[:, 1::2] = np.cos(position * div_term)
    return jnp.asarray(pe.astype(jnp.bfloat16))


def _charge_tc_body(ids_ref, tab_ref, pe_ref, out_ref):
    l = pl.program_id(0)
    b = pl.program_id(1)
    ids = ids_ref[pl.ds(b * L + l * TC_BLK, TC_BLK)]
    onehot = (ids[:, None] == lax.broadcasted_iota(jnp.int32, (TC_BLK, 256), 1))
    onehot = onehot.astype(jnp.float32)
    rows = jax.lax.dot_general(
        onehot, tab_ref[...],
        dimension_numbers=(((1,), (0,)), ((), ())),
        preferred_element_type=jnp.float32)
    out_ref[...] = rows + pe_ref[...].astype(jnp.float32)


def _charge_tc(ids, charge_table, pe):
    return pl.pallas_call(
        _charge_tc_body,
        grid=(N_LBLK, B),
        in_specs=[
            pl.BlockSpec((N_TOK,), lambda l, b: (0,)),
            pl.BlockSpec((256, D_MODEL), lambda l, b: (0, 0)),
            pl.BlockSpec((TC_BLK, D_MODEL), lambda l, b: (l, 0)),
        ],
        out_specs=pl.BlockSpec((TC_BLK, D_MODEL), lambda l, b: (b * N_LBLK + l, 0)),
        out_shape=jax.ShapeDtypeStruct((N_TOK, D_MODEL), jnp.float32),
    )(ids, charge_table, pe)


def _sc_body(ids_hbm, pos_hbm, mass_hbm,
             pos_out, mass_out,
             idx_v, ptab_v, pos_v, mtab_v, mass_v):
    wid = lax.axis_index("s") * NC + lax.axis_index("c")
    w128 = wid * L_PER_W

    for b in range(B):
        pltpu.sync_copy(ids_hbm.at[pl.ds(b * L + w128, L_PER_W)],
                        idx_v.at[pl.ds(b * L_PER_W, L_PER_W)])
    pltpu.sync_copy(pos_hbm, ptab_v)
    pltpu.sync_copy(mass_hbm, mtab_v)

    def _pos_step(j, carry):
        ids16 = idx_v[pl.ds(j * LANES, LANES)]
        flat_base = ids16 * POS_DIM
        row0 = (j // 8) * POS_DIM
        col = (j % 8) * LANES
        for c in range(POS_DIM):
            vals = plsc.load_gather(ptab_v, [flat_base + c])
            pos_v[row0 + c, pl.ds(col, LANES)] = vals
        return carry

    lax.fori_loop(0, TOK_PER_W // LANES, _pos_step, 0)
    for b in range(B):
        pltpu.sync_copy(
            pos_v.at[pl.ds(b * POS_DIM, POS_DIM)],
            pos_out.at[pl.ds(b * POS_DIM, POS_DIM), pl.ds(w128, L_PER_W)])

    def _mass_step(j, carry):
        ids16 = idx_v[pl.ds(j * LANES, LANES)]
        x = plsc.load_gather(mtab_v, [ids16])
        mass_v[pl.ds(j * LANES, LANES)] = 1.0 / (1.0 + jnp.exp(-x))
        return carry

    lax.fori_loop(0, TOK_PER_W // LANES, _mass_step, 0)
    for b in range(B):
        pltpu.sync_copy(mass_v.at[pl.ds(b * L_PER_W, L_PER_W)],
                        mass_out.at[pl.ds(b * L + w128, L_PER_W)])


@functools.partial(
    pl.kernel,
    out_type=[
        jax.ShapeDtypeStruct((B * POS_DIM, L), jnp.float32),
        jax.ShapeDtypeStruct((N_TOK,), jnp.float32),
    ],
    scratch_types=[
        pltpu.VMEM((TOK_PER_W,), jnp.int32),
        pltpu.VMEM((256 * POS_DIM,), jnp.float32),
        pltpu.VMEM((B * POS_DIM, L_PER_W), jnp.float32),
        pltpu.VMEM((256,), jnp.float32),
        pltpu.VMEM((TOK_PER_W,), jnp.float32),
    ],
    mesh=plsc.VectorSubcoreMesh(core_axis_name="c", subcore_axis_name="s"),
    compiler_params=pltpu.CompilerParams(needs_layout_passes=False),
)
def _pos_mass_sc(*args):
    _sc_body(*args)


def kernel(byte_ids, charge_table, position_table, mass_table):
    assert byte_ids.shape == (B, L)
    ids_flat = byte_ids.reshape(N_TOK).astype(jnp.int32)
    pe = _pe_table()
    charge_f = _charge_tc(ids_flat, charge_table, pe)
    pos_f, mass_f = _pos_mass_sc(
        ids_flat, position_table.reshape(256 * POS_DIM), mass_table.reshape(256))
    return (
        charge_f.reshape(B, L, D_MODEL),
        jnp.transpose(pos_f.reshape(B, POS_DIM, L), (0, 2, 1)),
        mass_f.reshape(B, L, 1),
    )
